# baseline (speedup 1.0000x reference)
import functools

import jax
import jax.numpy as jnp
from jax import lax
from jax.experimental import pallas as pl
from jax.experimental.pallas import tpu as pltpu
from jax.experimental.pallas import tpu_sc as plsc

N = 10000
E = 320000
C = 128
K = 27
EPS = 1e-5

CH = 128
NWORKERS = 32
NJ = 80
SG = 8
NSG = NJ // SG
E_PAD = NWORKERS * NJ * CH
NPAD = 10240
ROWS_PER_TILE = NPAD // 16


def _mm_body(x_ref, w_ref, o_ref):
    o_ref[...] = jnp.dot(x_ref[...], w_ref[...], preferred_element_type=jnp.float32)


def _transform(feats, W2):
    BN = 400
    return pl.pallas_call(
        _mm_body,
        grid=(N // BN,),
        in_specs=[
            pl.BlockSpec((BN, C), lambda i: (i, 0)),
            pl.BlockSpec((C, K * C), lambda i: (0, 0)),
        ],
        out_specs=pl.BlockSpec((BN, K * C), lambda i: (i, 0)),
        out_shape=jax.ShapeDtypeStruct((N, K * C), jnp.float32),
    )(feats, W2)


_MESH = plsc.VectorSubcoreMesh(core_axis_name="c", subcore_axis_name="s")


@functools.partial(
    pl.kernel,
    out_type=jax.ShapeDtypeStruct((2, NPAD, C), jnp.float32),
    mesh=_MESH,
    scratch_types=[
        pltpu.VMEM((2, SG, CH), jnp.int32),
        pltpu.VMEM((2, SG, CH), jnp.int32),
        pltpu.VMEM((2, CH, C), jnp.float32),
        pltpu.VMEM_SHARED((NPAD, C), jnp.float32),
        pltpu.SemaphoreType.DMA,
        pltpu.SemaphoreType.DMA,
        pltpu.SemaphoreType.DMA,
        pltpu.SemaphoreType.DMA,
        pltpu.SemaphoreType.DMA,
        pltpu.SemaphoreType.DMA,
    ],
)
def _sc_scatter(rid_hbm, dst_hbm, t_hbm, zeros_hbm, out_hbm,
                rid_v, dst_v, rows_v, acc_sh,
                sem_rid, sem_dst, g0, g1, s0, s1):
    gsem = (g0, g1)
    ssem = (s0, s1)
    cid = lax.axis_index("c")
    sid = lax.axis_index("s")
    wid = sid * 2 + cid
    base = wid * NJ

    def stage_fire(sg, p):
        sl = pl.ds(base + sg * SG, SG)
        pltpu.async_copy(rid_hbm.at[sl], rid_v.at[p], sem_rid)
        pltpu.async_copy(dst_hbm.at[sl], dst_v.at[p], sem_dst)

    def stage_wait(sg, p):
        sl = pl.ds(base + sg * SG, SG)
        pltpu.make_async_copy(rid_hbm.at[sl], rid_v.at[p], sem_rid).wait()
        pltpu.make_async_copy(dst_hbm.at[sl], dst_v.at[p], sem_dst).wait()

    def gather_fire(p, c, b):
        pltpu.async_copy(t_hbm.at[rid_v.at[p, c]], rows_v.at[b], gsem[b])

    def gather_wait(p, c, b):
        pltpu.make_async_copy(t_hbm.at[rid_v.at[p, c]], rows_v.at[b],
                              gsem[b]).wait()

    def scatter_fire(p, c, b):
        pltpu.async_copy(rows_v.at[b], acc_sh.at[dst_v.at[p, c]], ssem[b],
                         add=True)

    def scatter_wait(p, c, b):
        pltpu.make_async_copy(rows_v.at[b], acc_sh.at[dst_v.at[p, c]],
                              ssem[b]).wait()

    stage_fire(0, 0)
    pltpu.sync_copy(zeros_hbm.at[pl.ds(sid * ROWS_PER_TILE, ROWS_PER_TILE)],
                    acc_sh.at[pl.ds(sid * ROWS_PER_TILE, ROWS_PER_TILE)])
    plsc.subcore_barrier()

    for jj in range(NJ):
        sg, c = divmod(jj, SG)
        p = sg & 1
        b = jj & 1
        if c == 0:
            stage_wait(sg, p)
        if jj >= 2:
            pj, cj = divmod(jj - 2, SG)
            scatter_wait(pj & 1, cj, b)
        gather_fire(p, c, b)
        if jj >= 1:
            pj, cj = divmod(jj - 1, SG)
            gather_wait(pj & 1, cj, b ^ 1)
            scatter_fire(pj & 1, cj, b ^ 1)
        if c == 2 and sg + 1 < NSG:
            stage_fire(sg + 1, p ^ 1)
    gather_wait((NSG - 1) & 1, SG - 1, (NJ - 1) & 1)
    scatter_fire((NSG - 1) & 1, SG - 1, (NJ - 1) & 1)
    scatter_wait((NSG - 1) & 1, SG - 2, (NJ - 2) & 1)
    scatter_wait((NSG - 1) & 1, SG - 1, (NJ - 1) & 1)
    plsc.subcore_barrier()

    pltpu.sync_copy(acc_sh.at[pl.ds(sid * ROWS_PER_TILE, ROWS_PER_TILE)],
                    out_hbm.at[cid, pl.ds(sid * ROWS_PER_TILE, ROWS_PER_TILE)])


def _epi_body(p_ref, cb_ref, wl_ref, bl_ref, g_ref, b_ref, o_ref):
    conv = p_ref[0] + p_ref[1] + cb_ref[...]
    lin = lax.dot_general(conv, wl_ref[...], (((1,), (1,)), ((), ())),
                          preferred_element_type=jnp.float32) + bl_ref[...]
    mean = jnp.mean(lin, axis=1, keepdims=True)
    cent = lin - mean
    var = jnp.mean(cent * cent, axis=1, keepdims=True)
    o_ref[...] = cent * lax.rsqrt(var + EPS) * g_ref[...] + b_ref[...]


def _epilogue(partials, conv_bias, W_lin, b_lin, ln_gamma, ln_beta):
    BN = 1000
    return pl.pallas_call(
        _epi_body,
        grid=(N // BN,),
        in_specs=[
            pl.BlockSpec((2, BN, C), lambda i: (0, i, 0)),
            pl.BlockSpec((1, C), lambda i: (0, 0)),
            pl.BlockSpec((C, C), lambda i: (0, 0)),
            pl.BlockSpec((1, C), lambda i: (0, 0)),
            pl.BlockSpec((1, C), lambda i: (0, 0)),
            pl.BlockSpec((1, C), lambda i: (0, 0)),
        ],
        out_specs=pl.BlockSpec((BN, C), lambda i: (i, 0)),
        out_shape=jax.ShapeDtypeStruct((N, C), jnp.float32),
    )(partials, conv_bias.reshape(1, C), W_lin, b_lin.reshape(1, C),
      ln_gamma.reshape(1, C), ln_beta.reshape(1, C))


def kernel(feats, edge_index, edge_kernel, W_conv, conv_bias, W_lin, b_lin,
           ln_gamma, ln_beta):
    W2 = W_conv.transpose(1, 0, 2).reshape(C, K * C)
    T = _transform(feats, W2).reshape(N * K, C)
    zeros = jnp.zeros((NPAD, C), dtype=jnp.float32)
    pad = E_PAD - E
    rid = jnp.concatenate(
        [edge_index[0] * K + edge_kernel,
         jnp.zeros((pad,), jnp.int32)]).reshape(E_PAD // CH, CH)
    dst_p = jnp.concatenate(
        [edge_index[1],
         jnp.full((pad,), NPAD - 1, jnp.int32)]).reshape(E_PAD // CH, CH)
    partials = _sc_scatter(rid, dst_p, T, zeros)
    return _epilogue(partials, conv_bias, W_lin, b_lin, ln_gamma, ln_beta)


# speedup vs baseline: 1.8013x; 1.8013x over previous
import functools

import jax
import jax.numpy as jnp
from jax import lax
from jax.experimental import pallas as pl
from jax.experimental.pallas import tpu as pltpu
from jax.experimental.pallas import tpu_sc as plsc

N = 10000
E = 320000
C = 128
K = 27
EPS = 1e-5

CH = 128
NWORKERS = 32
NJ = 80
SG = 8
NSG = NJ // SG
E_PAD = NWORKERS * NJ * CH
NPAD = 10240
ROWS_PER_TILE = NPAD // 16


def _mm_body(x_ref, w_ref, o_ref):
    o_ref[...] = jnp.dot(x_ref[...], w_ref[...], preferred_element_type=jnp.float32)


def _transform(feats, W2):
    BN = 400
    return pl.pallas_call(
        _mm_body,
        grid=(N // BN,),
        in_specs=[
            pl.BlockSpec((BN, C), lambda i: (i, 0)),
            pl.BlockSpec((C, K * C), lambda i: (0, 0)),
        ],
        out_specs=pl.BlockSpec((BN, K * C), lambda i: (i, 0)),
        out_shape=jax.ShapeDtypeStruct((N, K * C), jnp.float32),
    )(feats, W2)


_MESH = plsc.VectorSubcoreMesh(core_axis_name="c", subcore_axis_name="s")


@functools.partial(
    pl.kernel,
    out_type=jax.ShapeDtypeStruct((2, NPAD, C), jnp.float32),
    mesh=_MESH,
    scratch_types=[
        pltpu.VMEM((2, SG, CH), jnp.int32),
        pltpu.VMEM((2, SG, CH), jnp.int32),
        pltpu.VMEM((2, CH, C), jnp.float32),
        pltpu.VMEM_SHARED((NPAD, C), jnp.float32),
        pltpu.SemaphoreType.DMA,
        pltpu.SemaphoreType.DMA,
        pltpu.SemaphoreType.DMA,
        pltpu.SemaphoreType.DMA,
        pltpu.SemaphoreType.DMA,
        pltpu.SemaphoreType.DMA,
    ],
)
def _sc_scatter(rid_hbm, dst_hbm, t_hbm, zeros_hbm, out_hbm,
                rid_v, dst_v, rows_v, acc_sh,
                sem_rid, sem_dst, g0, g1, s0, s1):
    gsem = (g0, g1)
    ssem = (s0, s1)
    cid = lax.axis_index("c")
    sid = lax.axis_index("s")
    wid = sid * 2 + cid
    base = wid * NJ

    def stage_fire(sg, p):
        sl = pl.ds(base + sg * SG, SG)
        pltpu.async_copy(rid_hbm.at[sl], rid_v.at[p], sem_rid)
        pltpu.async_copy(dst_hbm.at[sl], dst_v.at[p], sem_dst)

    def stage_wait(sg, p):
        sl = pl.ds(base + sg * SG, SG)
        pltpu.make_async_copy(rid_hbm.at[sl], rid_v.at[p], sem_rid).wait()
        pltpu.make_async_copy(dst_hbm.at[sl], dst_v.at[p], sem_dst).wait()

    def gather_fire(p, c, b):
        pltpu.async_copy(t_hbm.at[rid_v.at[p, c]], rows_v.at[b], gsem[b])

    def gather_wait(p, c, b):
        pltpu.make_async_copy(t_hbm.at[rid_v.at[p, c]], rows_v.at[b],
                              gsem[b]).wait()

    def scatter_fire(p, c, b):
        pltpu.async_copy(rows_v.at[b], acc_sh.at[dst_v.at[p, c]], ssem[b],
                         add=True)

    def scatter_wait(p, c, b):
        pltpu.make_async_copy(rows_v.at[b], acc_sh.at[dst_v.at[p, c]],
                              ssem[b]).wait()

    stage_fire(0, 0)
    pltpu.sync_copy(zeros_hbm.at[pl.ds(sid * ROWS_PER_TILE, ROWS_PER_TILE)],
                    acc_sh.at[pl.ds(sid * ROWS_PER_TILE, ROWS_PER_TILE)])
    plsc.subcore_barrier()

    for jj in range(NJ):
        sg, c = divmod(jj, SG)
        p = sg & 1
        b = jj & 1
        if c == 0:
            stage_wait(sg, p)
        if jj >= 2:
            pj, cj = divmod(jj - 2, SG)
            scatter_wait(pj & 1, cj, b)
        gather_fire(p, c, b)
        if jj >= 1:
            pj, cj = divmod(jj - 1, SG)
            gather_wait(pj & 1, cj, b ^ 1)
            scatter_fire(pj & 1, cj, b ^ 1)
        if c == 2 and sg + 1 < NSG:
            stage_fire(sg + 1, p ^ 1)
    gather_wait((NSG - 1) & 1, SG - 1, (NJ - 1) & 1)
    scatter_fire((NSG - 1) & 1, SG - 1, (NJ - 1) & 1)
    scatter_wait((NSG - 1) & 1, SG - 2, (NJ - 2) & 1)
    scatter_wait((NSG - 1) & 1, SG - 1, (NJ - 1) & 1)
    plsc.subcore_barrier()

    pltpu.sync_copy(acc_sh.at[pl.ds(sid * ROWS_PER_TILE, ROWS_PER_TILE)],
                    out_hbm.at[cid, pl.ds(sid * ROWS_PER_TILE, ROWS_PER_TILE)])


def _epi_body(p_ref, cb_ref, wl_ref, bl_ref, g_ref, b_ref, o_ref):
    conv = p_ref[0] + p_ref[1] + cb_ref[...]
    lin = lax.dot_general(conv, wl_ref[...], (((1,), (1,)), ((), ())),
                          preferred_element_type=jnp.float32) + bl_ref[...]
    mean = jnp.mean(lin, axis=1, keepdims=True)
    cent = lin - mean
    var = jnp.mean(cent * cent, axis=1, keepdims=True)
    o_ref[...] = cent * lax.rsqrt(var + EPS) * g_ref[...] + b_ref[...]


def _epilogue(partials, conv_bias, W_lin, b_lin, ln_gamma, ln_beta):
    BN = 1000
    return pl.pallas_call(
        _epi_body,
        grid=(N // BN,),
        in_specs=[
            pl.BlockSpec((2, BN, C), lambda i: (0, i, 0)),
            pl.BlockSpec((1, C), lambda i: (0, 0)),
            pl.BlockSpec((C, C), lambda i: (0, 0)),
            pl.BlockSpec((1, C), lambda i: (0, 0)),
            pl.BlockSpec((1, C), lambda i: (0, 0)),
            pl.BlockSpec((1, C), lambda i: (0, 0)),
        ],
        out_specs=pl.BlockSpec((BN, C), lambda i: (i, 0)),
        out_shape=jax.ShapeDtypeStruct((N, C), jnp.float32),
    )(partials, conv_bias.reshape(1, C), W_lin, b_lin.reshape(1, C),
      ln_gamma.reshape(1, C), ln_beta.reshape(1, C))


def kernel(feats, edge_index, edge_kernel, W_conv, conv_bias, W_lin, b_lin,
           ln_gamma, ln_beta):
    W2 = W_conv.transpose(1, 0, 2).reshape(C, K * C)
    T = _transform(feats, W2).reshape(N * K, C)
    zeros = jnp.zeros((NPAD, C), dtype=jnp.float32)
    pad = E_PAD - E
    pad_iota = jnp.arange(pad, dtype=jnp.int32)
    rid = jnp.concatenate(
        [edge_index[0] * K + edge_kernel,
         pad_iota % (N * K)]).reshape(E_PAD // CH, CH)
    dst_p = jnp.concatenate(
        [edge_index[1],
         N + pad_iota % (NPAD - N)]).reshape(E_PAD // CH, CH)
    partials = _sc_scatter(rid, dst_p, T, zeros)
    return _epilogue(partials, conv_bias, W_lin, b_lin, ln_gamma, ln_beta)


# speedup vs baseline: 2.9833x; 1.6562x over previous
import functools

import jax
import jax.numpy as jnp
from jax import lax
from jax.experimental import pallas as pl
from jax.experimental.pallas import tpu as pltpu
from jax.experimental.pallas import tpu_sc as plsc

N = 10000
E = 320000
C = 128
K = 27
EPS = 1e-5

CH = 128
NWORKERS = 32
NJ = 80
SG = 8
NSG = NJ // SG
E_PAD = NWORKERS * NJ * CH
NPAD = 10240
ROWS_PER_TILE = NPAD // 16


def _mm_body(x_ref, w_ref, o_ref):
    x = x_ref[...]
    for k in range(K):
        o_ref[k] = jnp.dot(x, w_ref[k], preferred_element_type=jnp.float32)


def _transform(feats, W_conv):
    BN = 400
    return pl.pallas_call(
        _mm_body,
        grid=(N // BN,),
        in_specs=[
            pl.BlockSpec((BN, C), lambda i: (i, 0)),
            pl.BlockSpec((K, C, C), lambda i: (0, 0, 0)),
        ],
        out_specs=pl.BlockSpec((K, BN, C), lambda i: (0, i, 0)),
        out_shape=jax.ShapeDtypeStruct((K, N, C), jnp.float32),
    )(feats, W_conv)


_MESH = plsc.VectorSubcoreMesh(core_axis_name="c", subcore_axis_name="s")


@functools.partial(
    pl.kernel,
    out_type=jax.ShapeDtypeStruct((2, NPAD, C), jnp.float32),
    mesh=_MESH,
    scratch_types=[
        pltpu.VMEM((2, SG, CH), jnp.int32),
        pltpu.VMEM((2, SG, CH), jnp.int32),
        pltpu.VMEM((2, CH, C), jnp.float32),
        pltpu.VMEM_SHARED((NPAD, C), jnp.float32),
        pltpu.SemaphoreType.DMA,
        pltpu.SemaphoreType.DMA,
        pltpu.SemaphoreType.DMA,
        pltpu.SemaphoreType.DMA,
        pltpu.SemaphoreType.DMA,
        pltpu.SemaphoreType.DMA,
    ],
)
def _sc_scatter(rid_hbm, dst_hbm, t_hbm, zeros_hbm, out_hbm,
                rid_v, dst_v, rows_v, acc_sh,
                sem_rid, sem_dst, g0, g1, s0, s1):
    gsem = (g0, g1)
    ssem = (s0, s1)
    cid = lax.axis_index("c")
    sid = lax.axis_index("s")
    wid = sid * 2 + cid
    base = wid * NJ

    def stage_fire(sg, p):
        sl = pl.ds(base + sg * SG, SG)
        pltpu.async_copy(rid_hbm.at[sl], rid_v.at[p], sem_rid)
        pltpu.async_copy(dst_hbm.at[sl], dst_v.at[p], sem_dst)

    def stage_wait(sg, p):
        sl = pl.ds(base + sg * SG, SG)
        pltpu.make_async_copy(rid_hbm.at[sl], rid_v.at[p], sem_rid).wait()
        pltpu.make_async_copy(dst_hbm.at[sl], dst_v.at[p], sem_dst).wait()

    def gather_fire(p, c, b):
        pltpu.async_copy(t_hbm.at[rid_v.at[p, c]], rows_v.at[b], gsem[b])

    def gather_wait(p, c, b):
        pltpu.make_async_copy(t_hbm.at[rid_v.at[p, c]], rows_v.at[b],
                              gsem[b]).wait()

    def scatter_fire(p, c, b):
        pltpu.async_copy(rows_v.at[b], acc_sh.at[dst_v.at[p, c]], ssem[b],
                         add=True)

    def scatter_wait(p, c, b):
        pltpu.make_async_copy(rows_v.at[b], acc_sh.at[dst_v.at[p, c]],
                              ssem[b]).wait()

    stage_fire(0, 0)
    pltpu.sync_copy(zeros_hbm.at[pl.ds(sid * ROWS_PER_TILE, ROWS_PER_TILE)],
                    acc_sh.at[pl.ds(sid * ROWS_PER_TILE, ROWS_PER_TILE)])
    plsc.subcore_barrier()

    for jj in range(NJ):
        sg, c = divmod(jj, SG)
        p = sg & 1
        b = jj & 1
        if c == 0:
            stage_wait(sg, p)
        if jj >= 2:
            pj, cj = divmod(jj - 2, SG)
            scatter_wait(pj & 1, cj, b)
        gather_fire(p, c, b)
        if jj >= 1:
            pj, cj = divmod(jj - 1, SG)
            gather_wait(pj & 1, cj, b ^ 1)
            scatter_fire(pj & 1, cj, b ^ 1)
        if c == 2 and sg + 1 < NSG:
            stage_fire(sg + 1, p ^ 1)
    gather_wait((NSG - 1) & 1, SG - 1, (NJ - 1) & 1)
    scatter_fire((NSG - 1) & 1, SG - 1, (NJ - 1) & 1)
    scatter_wait((NSG - 1) & 1, SG - 2, (NJ - 2) & 1)
    scatter_wait((NSG - 1) & 1, SG - 1, (NJ - 1) & 1)
    plsc.subcore_barrier()

    pltpu.sync_copy(acc_sh.at[pl.ds(sid * ROWS_PER_TILE, ROWS_PER_TILE)],
                    out_hbm.at[cid, pl.ds(sid * ROWS_PER_TILE, ROWS_PER_TILE)])


def _epi_body(p_ref, cb_ref, wl_ref, bl_ref, g_ref, b_ref, o_ref):
    conv = p_ref[0] + p_ref[1] + cb_ref[...]
    lin = lax.dot_general(conv, wl_ref[...], (((1,), (1,)), ((), ())),
                          preferred_element_type=jnp.float32) + bl_ref[...]
    mean = jnp.mean(lin, axis=1, keepdims=True)
    cent = lin - mean
    var = jnp.mean(cent * cent, axis=1, keepdims=True)
    o_ref[...] = cent * lax.rsqrt(var + EPS) * g_ref[...] + b_ref[...]


def _epilogue(partials, conv_bias, W_lin, b_lin, ln_gamma, ln_beta):
    BN = 1000
    return pl.pallas_call(
        _epi_body,
        grid=(N // BN,),
        in_specs=[
            pl.BlockSpec((2, BN, C), lambda i: (0, i, 0)),
            pl.BlockSpec((1, C), lambda i: (0, 0)),
            pl.BlockSpec((C, C), lambda i: (0, 0)),
            pl.BlockSpec((1, C), lambda i: (0, 0)),
            pl.BlockSpec((1, C), lambda i: (0, 0)),
            pl.BlockSpec((1, C), lambda i: (0, 0)),
        ],
        out_specs=pl.BlockSpec((BN, C), lambda i: (i, 0)),
        out_shape=jax.ShapeDtypeStruct((N, C), jnp.float32),
    )(partials, conv_bias.reshape(1, C), W_lin, b_lin.reshape(1, C),
      ln_gamma.reshape(1, C), ln_beta.reshape(1, C))


def kernel(feats, edge_index, edge_kernel, W_conv, conv_bias, W_lin, b_lin,
           ln_gamma, ln_beta):
    T = _transform(feats, W_conv).reshape(K * N, C)
    zeros = jnp.zeros((NPAD, C), dtype=jnp.float32)
    pad = E_PAD - E
    pad_iota = jnp.arange(pad, dtype=jnp.int32)
    rid = jnp.concatenate(
        [edge_kernel * N + edge_index[0],
         pad_iota % (N * K)]).reshape(E_PAD // CH, CH)
    dst_p = jnp.concatenate(
        [edge_index[1],
         N + pad_iota % (NPAD - N)]).reshape(E_PAD // CH, CH)
    partials = _sc_scatter(rid, dst_p, T, zeros)
    return _epilogue(partials, conv_bias, W_lin, b_lin, ln_gamma, ln_beta)


# speedup vs baseline: 2.9949x; 1.0039x over previous
import functools

import jax
import jax.numpy as jnp
from jax import lax
from jax.experimental import pallas as pl
from jax.experimental.pallas import tpu as pltpu
from jax.experimental.pallas import tpu_sc as plsc

N = 10000
E = 320000
C = 128
K = 27
EPS = 1e-5

CH = 64
NWORKERS = 32
NJ = 160
NB = 4
LAG = 2
SG = 16
NSG = NJ // SG
E_PAD = NWORKERS * NJ * CH
NPAD = 10240
ROWS_PER_TILE = NPAD // 16


def _mm_body(x_ref, w_ref, o_ref):
    x = x_ref[...]
    for k in range(K):
        o_ref[k] = jnp.dot(x, w_ref[k], preferred_element_type=jnp.float32)


def _transform(feats, W_conv):
    BN = 400
    return pl.pallas_call(
        _mm_body,
        grid=(N // BN,),
        in_specs=[
            pl.BlockSpec((BN, C), lambda i: (i, 0)),
            pl.BlockSpec((K, C, C), lambda i: (0, 0, 0)),
        ],
        out_specs=pl.BlockSpec((K, BN, C), lambda i: (0, i, 0)),
        out_shape=jax.ShapeDtypeStruct((K, N, C), jnp.float32),
    )(feats, W_conv)


_MESH = plsc.VectorSubcoreMesh(core_axis_name="c", subcore_axis_name="s")


@functools.partial(
    pl.kernel,
    out_type=jax.ShapeDtypeStruct((2, NPAD, C), jnp.float32),
    mesh=_MESH,
    scratch_types=[
        pltpu.VMEM((2, SG, CH), jnp.int32),
        pltpu.VMEM((2, SG, CH), jnp.int32),
        pltpu.VMEM((NB, CH, C), jnp.float32),
        pltpu.VMEM_SHARED((NPAD, C), jnp.float32),
        pltpu.SemaphoreType.DMA,
        pltpu.SemaphoreType.DMA,
        pltpu.SemaphoreType.DMA,
        pltpu.SemaphoreType.DMA,
        pltpu.SemaphoreType.DMA,
        pltpu.SemaphoreType.DMA,
        pltpu.SemaphoreType.DMA,
        pltpu.SemaphoreType.DMA,
        pltpu.SemaphoreType.DMA,
        pltpu.SemaphoreType.DMA,
    ],
)
def _sc_scatter(rid_hbm, dst_hbm, t_hbm, zeros_hbm, out_hbm,
                rid_v, dst_v, rows_v, acc_sh,
                sem_rid, sem_dst, g0, g1, g2, g3, s0, s1, s2, s3):
    gsem = (g0, g1, g2, g3)
    ssem = (s0, s1, s2, s3)
    cid = lax.axis_index("c")
    sid = lax.axis_index("s")
    wid = sid * 2 + cid
    base = wid * NJ

    def stage_fire(sg, p):
        sl = pl.ds(base + sg * SG, SG)
        pltpu.async_copy(rid_hbm.at[sl], rid_v.at[p], sem_rid)
        pltpu.async_copy(dst_hbm.at[sl], dst_v.at[p], sem_dst)

    def stage_wait(sg, p):
        sl = pl.ds(base + sg * SG, SG)
        pltpu.make_async_copy(rid_hbm.at[sl], rid_v.at[p], sem_rid).wait()
        pltpu.make_async_copy(dst_hbm.at[sl], dst_v.at[p], sem_dst).wait()

    def gather_fire(p, c):
        pltpu.async_copy(t_hbm.at[rid_v.at[p, c]], rows_v.at[c % NB],
                         gsem[c % NB])

    def gather_wait(p, c):
        pltpu.make_async_copy(t_hbm.at[rid_v.at[p, c]], rows_v.at[c % NB],
                              gsem[c % NB]).wait()

    def scatter_fire(p, c):
        pltpu.async_copy(rows_v.at[c % NB], acc_sh.at[dst_v.at[p, c]],
                         ssem[c % NB], add=True)

    def scatter_wait(p, c):
        pltpu.make_async_copy(rows_v.at[c % NB], acc_sh.at[dst_v.at[p, c]],
                              ssem[c % NB]).wait()

    stage_fire(0, 0)
    pltpu.sync_copy(zeros_hbm.at[pl.ds(sid * ROWS_PER_TILE, ROWS_PER_TILE)],
                    acc_sh.at[pl.ds(sid * ROWS_PER_TILE, ROWS_PER_TILE)])
    plsc.subcore_barrier()

    def double_group(g, carry):
        for p in (0, 1):
            sg = g * 2 + p
            for c in range(SG):
                if c == 0:
                    stage_wait(sg, p)
                if c >= NB:
                    scatter_wait(p, c - NB)
                elif p == 1:
                    scatter_wait(0, SG - NB + c)
                else:
                    @pl.when(g > 0)
                    def _():
                        scatter_wait(1, SG - NB + c)
                gather_fire(p, c)
                if c >= LAG:
                    gather_wait(p, c - LAG)
                    scatter_fire(p, c - LAG)
                elif p == 1:
                    gather_wait(0, SG - LAG + c)
                    scatter_fire(0, SG - LAG + c)
                else:
                    @pl.when(g > 0)
                    def _():
                        gather_wait(1, SG - LAG + c)
                        scatter_fire(1, SG - LAG + c)
                if c == NB:
                    if p == 0:
                        stage_fire(sg + 1, 1)
                    else:
                        @pl.when(g < NSG // 2 - 1)
                        def _():
                            stage_fire(sg + 1, 0)
        return carry

    lax.fori_loop(0, NSG // 2, double_group, 0)
    for c in range(SG - LAG, SG):
        gather_wait(1, c)
        scatter_fire(1, c)
    for c in range(SG - NB, SG):
        scatter_wait(1, c)
    plsc.subcore_barrier()

    pltpu.sync_copy(acc_sh.at[pl.ds(sid * ROWS_PER_TILE, ROWS_PER_TILE)],
                    out_hbm.at[cid, pl.ds(sid * ROWS_PER_TILE, ROWS_PER_TILE)])


def _epi_body(p_ref, cb_ref, wl_ref, bl_ref, g_ref, b_ref, o_ref):
    conv = p_ref[0] + p_ref[1] + cb_ref[...]
    lin = lax.dot_general(conv, wl_ref[...], (((1,), (1,)), ((), ())),
                          preferred_element_type=jnp.float32) + bl_ref[...]
    mean = jnp.mean(lin, axis=1, keepdims=True)
    cent = lin - mean
    var = jnp.mean(cent * cent, axis=1, keepdims=True)
    o_ref[...] = cent * lax.rsqrt(var + EPS) * g_ref[...] + b_ref[...]


def _epilogue(partials, conv_bias, W_lin, b_lin, ln_gamma, ln_beta):
    BN = 1000
    return pl.pallas_call(
        _epi_body,
        grid=(N // BN,),
        in_specs=[
            pl.BlockSpec((2, BN, C), lambda i: (0, i, 0)),
            pl.BlockSpec((1, C), lambda i: (0, 0)),
            pl.BlockSpec((C, C), lambda i: (0, 0)),
            pl.BlockSpec((1, C), lambda i: (0, 0)),
            pl.BlockSpec((1, C), lambda i: (0, 0)),
            pl.BlockSpec((1, C), lambda i: (0, 0)),
        ],
        out_specs=pl.BlockSpec((BN, C), lambda i: (i, 0)),
        out_shape=jax.ShapeDtypeStruct((N, C), jnp.float32),
    )(partials, conv_bias.reshape(1, C), W_lin, b_lin.reshape(1, C),
      ln_gamma.reshape(1, C), ln_beta.reshape(1, C))


def kernel(feats, edge_index, edge_kernel, W_conv, conv_bias, W_lin, b_lin,
           ln_gamma, ln_beta):
    T = _transform(feats, W_conv).reshape(K * N, C)
    zeros = jnp.zeros((NPAD, C), dtype=jnp.float32)
    pad = E_PAD - E
    pad_iota = jnp.arange(pad, dtype=jnp.int32)
    rid = jnp.concatenate(
        [edge_kernel * N + edge_index[0],
         pad_iota % (N * K)]).reshape(E_PAD // CH, CH)
    dst_p = jnp.concatenate(
        [edge_index[1],
         N + pad_iota % (NPAD - N)]).reshape(E_PAD // CH, CH)
    partials = _sc_scatter(rid, dst_p, T, zeros)
    return _epilogue(partials, conv_bias, W_lin, b_lin, ln_gamma, ln_beta)


# speedup vs baseline: 2.9994x; 1.0015x over previous
import functools

import jax
import jax.numpy as jnp
from jax import lax
from jax.experimental import pallas as pl
from jax.experimental.pallas import tpu as pltpu
from jax.experimental.pallas import tpu_sc as plsc

N = 10000
E = 320000
C = 128
K = 27
EPS = 1e-5

CH = 64
NWORKERS = 32
NJ = 160
NB = 4
LAG = 2
SG = 16
NSG = NJ // SG
E_PAD = NWORKERS * NJ * CH
NPAD = 10240
ROWS_PER_TILE = NPAD // 16


def _mm_body(x_ref, w_ref, o_ref):
    x = x_ref[...]
    for k in range(K):
        o_ref[k] = jnp.dot(x, w_ref[k], preferred_element_type=jnp.float32)


def _transform(feats, W_conv):
    BN = 400
    return pl.pallas_call(
        _mm_body,
        grid=(N // BN,),
        in_specs=[
            pl.BlockSpec((BN, C), lambda i: (i, 0)),
            pl.BlockSpec((K, C, C), lambda i: (0, 0, 0)),
        ],
        out_specs=pl.BlockSpec((K, BN, C), lambda i: (0, i, 0)),
        out_shape=jax.ShapeDtypeStruct((K, N, C), jnp.float32),
    )(feats, W_conv)


_MESH = plsc.VectorSubcoreMesh(core_axis_name="c", subcore_axis_name="s")


@functools.partial(
    pl.kernel,
    out_type=jax.ShapeDtypeStruct((2, NPAD, C), jnp.float32),
    mesh=_MESH,
    scratch_types=[
        pltpu.VMEM((2, SG, CH), jnp.int32),
        pltpu.VMEM((2, SG, CH), jnp.int32),
        pltpu.VMEM((NB, CH, C), jnp.float32),
        pltpu.VMEM_SHARED((NPAD, C), jnp.float32),
        pltpu.SemaphoreType.DMA,
        pltpu.SemaphoreType.DMA,
        pltpu.SemaphoreType.DMA,
        pltpu.SemaphoreType.DMA,
        pltpu.SemaphoreType.DMA,
        pltpu.SemaphoreType.DMA,
        pltpu.SemaphoreType.DMA,
        pltpu.SemaphoreType.DMA,
        pltpu.SemaphoreType.DMA,
        pltpu.SemaphoreType.DMA,
    ],
)
def _sc_scatter(rid_hbm, dst_hbm, t_hbm, zeros_hbm, out_hbm,
                rid_v, dst_v, rows_v, acc_sh,
                sem_rid, sem_dst, g0, g1, g2, g3, s0, s1, s2, s3):
    gsem = (g0, g1, g2, g3)
    ssem = (s0, s1, s2, s3)
    cid = lax.axis_index("c")
    sid = lax.axis_index("s")
    wid = sid * 2 + cid
    base = wid * NJ

    def stage_fire(sg, p):
        sl = pl.ds(base + sg * SG, SG)
        pltpu.async_copy(rid_hbm.at[sl], rid_v.at[p], sem_rid)
        pltpu.async_copy(dst_hbm.at[sl], dst_v.at[p], sem_dst)

    def stage_wait(sg, p):
        sl = pl.ds(base + sg * SG, SG)
        pltpu.make_async_copy(rid_hbm.at[sl], rid_v.at[p], sem_rid).wait()
        pltpu.make_async_copy(dst_hbm.at[sl], dst_v.at[p], sem_dst).wait()

    def gather_fire(p, c):
        pltpu.async_copy(t_hbm.at[rid_v.at[p, c]], rows_v.at[c % NB],
                         gsem[c % NB])

    def gather_wait(p, c):
        pltpu.make_async_copy(t_hbm.at[rid_v.at[p, c]], rows_v.at[c % NB],
                              gsem[c % NB]).wait()

    def scatter_fire(p, c):
        pltpu.async_copy(rows_v.at[c % NB], acc_sh.at[dst_v.at[p, c]],
                         ssem[c % NB], add=True)

    def scatter_wait(p, c):
        pltpu.make_async_copy(rows_v.at[c % NB], acc_sh.at[dst_v.at[p, c]],
                              ssem[c % NB]).wait()

    stage_fire(0, 0)
    pltpu.sync_copy(zeros_hbm,
                    acc_sh.at[pl.ds(sid * ROWS_PER_TILE, ROWS_PER_TILE)])
    plsc.subcore_barrier()

    def double_group(g, carry):
        for p in (0, 1):
            sg = g * 2 + p
            for c in range(SG):
                if c == 0:
                    stage_wait(sg, p)
                if c >= NB:
                    scatter_wait(p, c - NB)
                elif p == 1:
                    scatter_wait(0, SG - NB + c)
                else:
                    @pl.when(g > 0)
                    def _():
                        scatter_wait(1, SG - NB + c)
                gather_fire(p, c)
                if c >= LAG:
                    gather_wait(p, c - LAG)
                    scatter_fire(p, c - LAG)
                elif p == 1:
                    gather_wait(0, SG - LAG + c)
                    scatter_fire(0, SG - LAG + c)
                else:
                    @pl.when(g > 0)
                    def _():
                        gather_wait(1, SG - LAG + c)
                        scatter_fire(1, SG - LAG + c)
                if c == NB:
                    if p == 0:
                        stage_fire(sg + 1, 1)
                    else:
                        @pl.when(g < NSG // 2 - 1)
                        def _():
                            stage_fire(sg + 1, 0)
        return carry

    lax.fori_loop(0, NSG // 2, double_group, 0)
    for c in range(SG - LAG, SG):
        gather_wait(1, c)
        scatter_fire(1, c)
    for c in range(SG - NB, SG):
        scatter_wait(1, c)
    plsc.subcore_barrier()

    pltpu.sync_copy(acc_sh.at[pl.ds(sid * ROWS_PER_TILE, ROWS_PER_TILE)],
                    out_hbm.at[cid, pl.ds(sid * ROWS_PER_TILE, ROWS_PER_TILE)])


def _epi_body(p_ref, cb_ref, wl_ref, bl_ref, g_ref, b_ref, o_ref):
    conv = p_ref[0] + p_ref[1] + cb_ref[...]
    lin = lax.dot_general(conv, wl_ref[...], (((1,), (1,)), ((), ())),
                          preferred_element_type=jnp.float32) + bl_ref[...]
    mean = jnp.mean(lin, axis=1, keepdims=True)
    cent = lin - mean
    var = jnp.mean(cent * cent, axis=1, keepdims=True)
    o_ref[...] = cent * lax.rsqrt(var + EPS) * g_ref[...] + b_ref[...]


def _epilogue(partials, conv_bias, W_lin, b_lin, ln_gamma, ln_beta):
    BN = 1000
    return pl.pallas_call(
        _epi_body,
        grid=(N // BN,),
        in_specs=[
            pl.BlockSpec((2, BN, C), lambda i: (0, i, 0)),
            pl.BlockSpec((1, C), lambda i: (0, 0)),
            pl.BlockSpec((C, C), lambda i: (0, 0)),
            pl.BlockSpec((1, C), lambda i: (0, 0)),
            pl.BlockSpec((1, C), lambda i: (0, 0)),
            pl.BlockSpec((1, C), lambda i: (0, 0)),
        ],
        out_specs=pl.BlockSpec((BN, C), lambda i: (i, 0)),
        out_shape=jax.ShapeDtypeStruct((N, C), jnp.float32),
    )(partials, conv_bias.reshape(1, C), W_lin, b_lin.reshape(1, C),
      ln_gamma.reshape(1, C), ln_beta.reshape(1, C))


def kernel(feats, edge_index, edge_kernel, W_conv, conv_bias, W_lin, b_lin,
           ln_gamma, ln_beta):
    T = _transform(feats, W_conv).reshape(K * N, C)
    zeros = jnp.zeros((ROWS_PER_TILE, C), dtype=jnp.float32)
    pad = E_PAD - E
    pad_iota = jnp.arange(pad, dtype=jnp.int32)
    rid = jnp.concatenate(
        [edge_kernel * N + edge_index[0],
         pad_iota]).reshape(E_PAD // CH, CH)
    dst_p = jnp.concatenate(
        [edge_index[1],
         N + (pad_iota & 127)]).reshape(E_PAD // CH, CH)
    partials = _sc_scatter(rid, dst_p, T, zeros)
    return _epilogue(partials, conv_bias, W_lin, b_lin, ln_gamma, ln_beta)
